# baseline (speedup 1.0000x reference)
import functools

import numpy as np
import jax
import jax.numpy as jnp
from jax.experimental import pallas as pl
from jax.experimental.pallas import tpu as pltpu


def _pool_matrix(n_in: int, n_out: int) -> np.ndarray:
    m = np.zeros((n_out, n_in), dtype=np.float32)
    for i in range(n_out):
        start = (i * n_in) // n_out
        end = -((-(i + 1) * n_in) // n_out)
        m[i, start:end] = 1.0 / float(end - start)
    return m


def _pick_tile(B: int, target: int = 256) -> int:
    if B <= 8:
        return B
    tb = min(target, (B // 8) * 8)
    tb = (tb // 8) * 8
    while tb >= 8:
        if B % tb == 0:
            return tb
        tb -= 8
    return B


def _pool_body(x_ref, p_ref, o_ref):
    o_ref[...] = jnp.dot(
        x_ref[...], p_ref[...], preferred_element_type=jnp.float32
    ).astype(o_ref.dtype)


@functools.partial(jax.jit, static_argnums=(1, 2))
def _adaptive_pool(x, H: int, W: int):
    B, N, E = x.shape
    K = N * E
    HW = H * W
    P = jnp.asarray(np.kron(_pool_matrix(N, H), _pool_matrix(E, W)).T)

    tb = _pick_tile(B)
    n_blocks = B // tb if B % tb == 0 else int(pl.cdiv(B, tb))
    x2 = x.reshape(B, K)

    cost = pl.CostEstimate(
        flops=2 * B * K * HW,
        transcendentals=0,
        bytes_accessed=B * K * 4 + K * HW * 4 + B * HW * 4,
    )
    return pl.pallas_call(
        _pool_body,
        out_shape=jax.ShapeDtypeStruct((B, HW), x.dtype),
        grid=(n_blocks,),
        in_specs=[
            pl.BlockSpec((tb, K), lambda b: (b, 0)),
            pl.BlockSpec((K, HW), lambda b: (0, 0)),
        ],
        out_specs=pl.BlockSpec((tb, HW), lambda b: (b, 0)),
        compiler_params=pltpu.CompilerParams(
            dimension_semantics=("parallel",),
        ),
        cost_estimate=cost,
    )(x2, P)


def kernel(x):
    return _adaptive_pool(x, 4, 8)


# speedup vs baseline: 1.0267x; 1.0267x over previous
import functools

import numpy as np
import jax
import jax.numpy as jnp
from jax.experimental import pallas as pl
from jax.experimental.pallas import tpu as pltpu


def _pool_matrix(n_in: int, n_out: int) -> np.ndarray:
    m = np.zeros((n_out, n_in), dtype=np.float32)
    for i in range(n_out):
        start = (i * n_in) // n_out
        end = -((-(i + 1) * n_in) // n_out)
        m[i, start:end] = 1.0 / float(end - start)
    return m


def _pick_tile(B: int, target: int = 512) -> int:
    if B <= 8:
        return B
    tb = min(target, (B // 8) * 8)
    tb = (tb // 8) * 8
    while tb >= 8:
        if B % tb == 0:
            return tb
        tb -= 8
    return B


def _pool_body(x_ref, p_ref, o_ref):
    o_ref[...] = jnp.dot(
        x_ref[...], p_ref[...], preferred_element_type=jnp.float32
    ).astype(o_ref.dtype)


@functools.partial(jax.jit, static_argnums=(1, 2))
def _adaptive_pool(x, H: int, W: int):
    B, N, E = x.shape
    K = N * E
    HW = H * W
    P = jnp.asarray(np.kron(_pool_matrix(N, H), _pool_matrix(E, W)).T)

    tb = _pick_tile(B)
    n_blocks = B // tb if B % tb == 0 else int(pl.cdiv(B, tb))
    x2 = x.reshape(B, K)

    cost = pl.CostEstimate(
        flops=2 * B * K * HW,
        transcendentals=0,
        bytes_accessed=B * K * 4 + K * HW * 4 + B * HW * 4,
    )
    return pl.pallas_call(
        _pool_body,
        out_shape=jax.ShapeDtypeStruct((B, HW), x.dtype),
        grid=(n_blocks,),
        in_specs=[
            pl.BlockSpec((tb, K), lambda b: (b, 0)),
            pl.BlockSpec((K, HW), lambda b: (0, 0)),
        ],
        out_specs=pl.BlockSpec((tb, HW), lambda b: (b, 0)),
        compiler_params=pltpu.CompilerParams(
            dimension_semantics=("parallel",),
        ),
        cost_estimate=cost,
    )(x2, P)


def kernel(x):
    return _adaptive_pool(x, 4, 8)
